# baseline (speedup 1.0000x reference)
import functools

import jax
import jax.numpy as jnp
from jax.experimental import pallas as pl
from jax.experimental.pallas import tpu as pltpu


def _leaky(x, slope=0.1):
    return jnp.where(x >= 0, x, slope * x)


def _gnn_kernel(NB, C, E, NI, H,
                x_ref, eidx_ref, etype_ref,
                w1_ref, b1_ref, w2_ref, b2_ref,
                wf_hbm, wt_hbm, wet_ref, bed_ref,
                wsec_ref, bsec_ref, out_ref,
                wf_buf, wt_buf, wf_sem, wt_sem):
    f32 = jnp.float32

    for i in range(NI):
        pltpu.make_async_copy(wf_hbm.at[i], wf_buf.at[i], wf_sem.at[i]).start()
        pltpu.make_async_copy(wt_hbm.at[i], wt_buf.at[i], wt_sem.at[i]).start()

    x = x_ref[...]
    h = jnp.dot(x, w1_ref[...], preferred_element_type=f32) + b1_ref[...]
    h = _leaky(h, 0.01)
    h = _leaky(jnp.dot(h, w2_ref[...],
                       preferred_element_type=f32) + b2_ref[...])
    out_acc = jnp.dot(h, wsec_ref[0], preferred_element_type=f32)

    lane = jax.lax.broadcasted_iota(jnp.int32, (2 * E, C), 1)
    ohs, scats = [], []
    for b in range(NB):
        eb = eidx_ref[b]
        gft = jnp.concatenate([eb[:, 0:1], eb[:, 1:2]], axis=0)
        oh = (lane == gft).astype(f32)
        ohs.append(oh)
        scats.append(oh[:E, :].T)

    etype = etype_ref[...]

    cur = h
    for i in range(NI):
        gs = [jnp.dot(ohs[b], cur[b * C:(b + 1) * C, :],
                      preferred_element_type=f32) for b in range(NB)]
        gf = jnp.concatenate([g[:E, :] for g in gs], axis=0)
        gt = jnp.concatenate([g[E:, :] for g in gs], axis=0)
        pltpu.make_async_copy(wf_hbm.at[i], wf_buf.at[i], wf_sem.at[i]).wait()
        pltpu.make_async_copy(wt_hbm.at[i], wt_buf.at[i], wt_sem.at[i]).wait()
        z = (jnp.dot(gf, wf_buf[i], preferred_element_type=f32)
             + jnp.dot(gt, wt_buf[i], preferred_element_type=f32)
             + jnp.dot(etype, wet_ref[i], preferred_element_type=f32))
        z = jnp.maximum(z + bed_ref[i], 0.0)
        cur = jnp.concatenate(
            [jnp.dot(scats[b], z[b * E:(b + 1) * E, :],
                     preferred_element_type=f32) for b in range(NB)], axis=0)
        out_acc = out_acc + jnp.dot(cur, wsec_ref[i + 1],
                                    preferred_element_type=f32)

    out_ref[...] = _leaky(out_acc + bsec_ref[...])


@jax.jit
def kernel(child_feats, edge_indices, edge_type_onehot, lengths,
           w1, b1, w2, b2, wf, wt, wet, bed, wsec, bsec):
    del lengths
    B, C, Fin = child_feats.shape
    E = edge_indices.shape[1]
    T = edge_type_onehot.shape[2]
    NI, _, H = wf.shape
    F_out = wsec.shape[2]
    f32 = jnp.float32

    x = child_feats.astype(f32).reshape(B * C, Fin)
    eidx = edge_indices.astype(jnp.int32)
    etype = edge_type_onehot.astype(f32).reshape(B * E, T)

    vmem = pl.BlockSpec(memory_space=pltpu.VMEM)
    hbm = pl.BlockSpec(memory_space=pl.ANY)

    kern = functools.partial(_gnn_kernel, B, C, E, NI, H)
    out = pl.pallas_call(
        kern,
        out_shape=jax.ShapeDtypeStruct((B * C, F_out), f32),
        in_specs=[vmem, vmem, vmem,
                  vmem, vmem, vmem, vmem,
                  hbm, hbm, vmem, vmem,
                  vmem, vmem],
        out_specs=vmem,
        scratch_shapes=[
            pltpu.VMEM((NI, H, H), f32),
            pltpu.VMEM((NI, H, H), f32),
            pltpu.SemaphoreType.DMA((NI,)),
            pltpu.SemaphoreType.DMA((NI,)),
        ],
    )(x, eidx, etype, w1, b1, w2, b2, wf, wt, wet, bed, wsec, bsec)
    return out.reshape(B, C, F_out)


# speedup vs baseline: 1.1333x; 1.1333x over previous
import functools

import jax
import jax.numpy as jnp
from jax.experimental import pallas as pl
from jax.experimental.pallas import tpu as pltpu


def _leaky(x, slope=0.1):
    return jnp.where(x >= 0, x, slope * x)


def _gnn_kernel(NB, C, E, NI, H,
                x_ref, eidx_ref, etype_ref,
                w1_ref, b1_ref, w2_ref, b2_ref,
                wf_hbm, wt_hbm, wet_ref, bed_ref,
                wsec_ref, bsec_ref, out_ref,
                wf_buf, wt_buf, wf_sem, wt_sem):
    f32 = jnp.float32
    bf16 = jnp.bfloat16

    for i in range(NI):
        pltpu.make_async_copy(wf_hbm.at[i], wf_buf.at[i], wf_sem.at[i]).start()
        pltpu.make_async_copy(wt_hbm.at[i], wt_buf.at[i], wt_sem.at[i]).start()

    x = x_ref[...]
    h = jnp.dot(x, w1_ref[...], preferred_element_type=f32) + b1_ref[...]
    h = _leaky(h, 0.01)
    h = _leaky(jnp.dot(h.astype(bf16), w2_ref[...].astype(bf16),
                       preferred_element_type=f32) + b2_ref[...])
    out_acc = jnp.dot(h.astype(bf16), wsec_ref[0].astype(bf16),
                      preferred_element_type=f32)

    lane = jax.lax.broadcasted_iota(jnp.int32, (2 * E, C), 1)
    ohs, scats = [], []
    for b in range(NB):
        eb = eidx_ref[b]
        gft = jnp.concatenate([eb[:, 0:1], eb[:, 1:2]], axis=0)
        oh = (lane == gft).astype(bf16)
        ohs.append(oh)
        scats.append(oh[:E, :].T)

    etype = etype_ref[...]

    cur = h
    for i in range(NI):
        curb = cur.astype(bf16)
        gs = [jnp.dot(ohs[b], curb[b * C:(b + 1) * C, :],
                      preferred_element_type=f32) for b in range(NB)]
        gf = jnp.concatenate([g[:E, :] for g in gs], axis=0)
        gt = jnp.concatenate([g[E:, :] for g in gs], axis=0)
        pltpu.make_async_copy(wf_hbm.at[i], wf_buf.at[i], wf_sem.at[i]).wait()
        pltpu.make_async_copy(wt_hbm.at[i], wt_buf.at[i], wt_sem.at[i]).wait()
        z = (jnp.dot(gf.astype(bf16), wf_buf[i].astype(bf16),
                     preferred_element_type=f32)
             + jnp.dot(gt.astype(bf16), wt_buf[i].astype(bf16),
                       preferred_element_type=f32)
             + jnp.dot(etype, wet_ref[i], preferred_element_type=f32))
        z = jnp.maximum(z + bed_ref[i], 0.0).astype(bf16)
        cur = jnp.concatenate(
            [jnp.dot(scats[b], z[b * E:(b + 1) * E, :],
                     preferred_element_type=f32) for b in range(NB)], axis=0)
        out_acc = out_acc + jnp.dot(cur.astype(bf16),
                                    wsec_ref[i + 1].astype(bf16),
                                    preferred_element_type=f32)

    out_ref[...] = _leaky(out_acc + bsec_ref[...])


@jax.jit
def kernel(child_feats, edge_indices, edge_type_onehot, lengths,
           w1, b1, w2, b2, wf, wt, wet, bed, wsec, bsec):
    del lengths
    B, C, Fin = child_feats.shape
    E = edge_indices.shape[1]
    T = edge_type_onehot.shape[2]
    NI, _, H = wf.shape
    F_out = wsec.shape[2]
    f32 = jnp.float32

    x = child_feats.astype(f32).reshape(B * C, Fin)
    eidx = edge_indices.astype(jnp.int32)
    etype = edge_type_onehot.astype(f32).reshape(B * E, T)

    vmem = pl.BlockSpec(memory_space=pltpu.VMEM)
    hbm = pl.BlockSpec(memory_space=pl.ANY)

    kern = functools.partial(_gnn_kernel, B, C, E, NI, H)
    out = pl.pallas_call(
        kern,
        out_shape=jax.ShapeDtypeStruct((B * C, F_out), f32),
        in_specs=[vmem, vmem, vmem,
                  vmem, vmem, vmem, vmem,
                  hbm, hbm, vmem, vmem,
                  vmem, vmem],
        out_specs=vmem,
        scratch_shapes=[
            pltpu.VMEM((NI, H, H), f32),
            pltpu.VMEM((NI, H, H), f32),
            pltpu.SemaphoreType.DMA((NI,)),
            pltpu.SemaphoreType.DMA((NI,)),
        ],
    )(x, eidx, etype, w1, b1, w2, b2, wf, wt, wet, bed, wsec, bsec)
    return out.reshape(B, C, F_out)
